# Initial kernel scaffold; baseline (speedup 1.0000x reference)
import jax
import jax.numpy as jnp
from jax.experimental import pallas as pl
from jax.experimental.pallas import tpu as pltpu

_NEG = -1e30


def _target_kernel(h_ref, w_ref, out_ref):
    out_ref[...] = jax.lax.dot_general(
        h_ref[...], w_ref[...], (((1,), (1,)), ((), ())),
        preferred_element_type=jnp.float32)


def _attn_kernel(lens_ref, sel_ref, ctx_ref, tgt_ref, attn_ref, wout_ref,
                 scl_ref, md_ref, acc_ref):
    b = pl.program_id(0)
    s = pl.program_id(1)
    ns = pl.num_programs(1)
    ts = ctx_ref.shape[1]
    nblk = lens_ref.shape[1]

    @pl.when(s == 0)
    def _init():
        sel = sel_ref[b]

        def body(j, tot):
            return tot + jnp.where(j < sel, lens_ref[b, j], 0)

        scl_ref[0] = jax.lax.fori_loop(0, nblk, body, 0) + 1
        scl_ref[1] = lens_ref[b, sel]
        md_ref[0] = _NEG
        md_ref[1] = 0.0
        acc_ref[...] = jnp.zeros_like(acc_ref)

    ctx = ctx_ref[0]
    tgt = tgt_ref[...]
    attn_row = jax.lax.dot_general(
        tgt, ctx, (((1,), (1,)), ((), ())),
        preferred_element_type=jnp.float32)
    attn_ref[...] = attn_row

    sc = scl_ref[0]
    ln = scl_ref[1]
    t0 = s * ts
    overlap = (t0 < sc + ln) & (t0 + ts > sc)

    @pl.when(overlap)
    def _update():
        pos = t0 + jax.lax.broadcasted_iota(jnp.int32, (1, ts), 1)
        mask = (pos >= sc) & (pos < sc + ln)
        masked = jnp.where(mask, attn_row, _NEG)
        m_old = md_ref[0]
        m_new = jnp.maximum(m_old, jnp.max(masked))
        scale = jnp.exp(m_old - m_new)
        unnorm = jnp.where(mask, jnp.exp(masked - m_new), 0.0)
        md_ref[0] = m_new
        md_ref[1] = md_ref[1] * scale + jnp.sum(unnorm)
        acc_ref[...] = acc_ref[...] * scale + jax.lax.dot_general(
            unnorm, ctx, (((1,), (0,)), ((), ())),
            preferred_element_type=jnp.float32)

    @pl.when(s == ns - 1)
    def _finalize():
        d = md_ref[1]
        denom = jnp.where(d == 0.0, 1.0, d)
        wout_ref[...] = acc_ref[...] / denom


def kernel(h, context, sub_seq_lengths, selected_block_idx, W):
    batch, dim = h.shape
    seq = context.shape[1]

    td = 512
    target = pl.pallas_call(
        _target_kernel,
        grid=(dim // td,),
        in_specs=[
            pl.BlockSpec((batch, dim), lambda i: (0, 0)),
            pl.BlockSpec((td, dim), lambda i: (i, 0)),
        ],
        out_specs=pl.BlockSpec((batch, td), lambda i: (0, i)),
        out_shape=jax.ShapeDtypeStruct((batch, dim), jnp.float32),
    )(h, W)

    ts = 256
    lens = sub_seq_lengths.astype(jnp.int32)
    sel = selected_block_idx.astype(jnp.int32)
    attn, weighted = pl.pallas_call(
        _attn_kernel,
        grid=(batch, seq // ts),
        in_specs=[
            pl.BlockSpec(memory_space=pltpu.SMEM),
            pl.BlockSpec(memory_space=pltpu.SMEM),
            pl.BlockSpec((1, ts, dim), lambda b, s: (b, s, 0)),
            pl.BlockSpec((1, dim), lambda b, s: (b, 0)),
        ],
        out_specs=[
            pl.BlockSpec((1, ts), lambda b, s: (b, s)),
            pl.BlockSpec((1, dim), lambda b, s: (b, 0)),
        ],
        out_shape=[
            jax.ShapeDtypeStruct((batch, seq), jnp.float32),
            jax.ShapeDtypeStruct((batch, dim), jnp.float32),
        ],
        scratch_shapes=[
            pltpu.SMEM((2,), jnp.int32),
            pltpu.SMEM((2,), jnp.float32),
            pltpu.VMEM((1, dim), jnp.float32),
        ],
    )(lens, sel, context, target)
    return (weighted, attn)


# speedup vs baseline: 1.4208x; 1.4208x over previous
import jax
import jax.numpy as jnp
from jax.experimental import pallas as pl
from jax.experimental.pallas import tpu as pltpu

_NEG = -1e30


def _target_kernel(h_ref, w_ref, out_ref):
    out_ref[...] = jax.lax.dot_general(
        h_ref[...], w_ref[...], (((1,), (1,)), ((), ())),
        preferred_element_type=jnp.float32)


def _attn_kernel(lens_ref, sel_ref, ctx_ref, tgt_ref, attn_ref, wout_ref,
                 scl_ref, md_ref, acc_ref):
    b = pl.program_id(0)
    s = pl.program_id(1)
    ns = pl.num_programs(1)
    ts = ctx_ref.shape[1]
    nblk = lens_ref.shape[1]

    @pl.when(s == 0)
    def _init():
        sel = sel_ref[b]

        def body(j, tot):
            return tot + jnp.where(j < sel, lens_ref[b, j], 0)

        scl_ref[0] = jax.lax.fori_loop(0, nblk, body, 0) + 1
        scl_ref[1] = lens_ref[b, sel]
        md_ref[0] = _NEG
        md_ref[1] = 0.0
        acc_ref[...] = jnp.zeros_like(acc_ref)

    ctx = ctx_ref[0]
    tgt = tgt_ref[0]
    attn_row = jax.lax.dot_general(
        tgt, ctx, (((1,), (1,)), ((), ())),
        preferred_element_type=jnp.float32)
    attn_ref[0] = attn_row

    sc = scl_ref[0]
    ln = scl_ref[1]
    t0 = s * ts
    overlap = (t0 < sc + ln) & (t0 + ts > sc)

    @pl.when(overlap)
    def _update():
        pos = t0 + jax.lax.broadcasted_iota(jnp.int32, (1, ts), 1)
        mask = (pos >= sc) & (pos < sc + ln)
        masked = jnp.where(mask, attn_row, _NEG)
        m_old = md_ref[0]
        m_new = jnp.maximum(m_old, jnp.max(masked))
        scale = jnp.exp(m_old - m_new)
        unnorm = jnp.where(mask, jnp.exp(masked - m_new), 0.0)
        md_ref[0] = m_new
        md_ref[1] = md_ref[1] * scale + jnp.sum(unnorm)
        acc_ref[...] = acc_ref[...] * scale + jax.lax.dot_general(
            unnorm, ctx, (((1,), (0,)), ((), ())),
            preferred_element_type=jnp.float32)

    @pl.when(s == ns - 1)
    def _finalize():
        d = md_ref[1]
        denom = jnp.where(d == 0.0, 1.0, d)
        wout_ref[0] = acc_ref[...] / denom


def kernel(h, context, sub_seq_lengths, selected_block_idx, W):
    batch, dim = h.shape
    seq = context.shape[1]

    td = 512
    target = pl.pallas_call(
        _target_kernel,
        grid=(dim // td,),
        in_specs=[
            pl.BlockSpec((batch, dim), lambda i: (0, 0)),
            pl.BlockSpec((td, dim), lambda i: (i, 0)),
        ],
        out_specs=pl.BlockSpec((batch, td), lambda i: (0, i)),
        out_shape=jax.ShapeDtypeStruct((batch, dim), jnp.float32),
    )(h, W)

    ts = 256
    ns = seq // ts
    lens = sub_seq_lengths.astype(jnp.int32)
    sel = selected_block_idx.astype(jnp.int32)
    tgt3 = target.reshape(batch, 1, dim)
    attn, weighted = pl.pallas_call(
        _attn_kernel,
        grid=(batch, ns),
        in_specs=[
            pl.BlockSpec(memory_space=pltpu.SMEM),
            pl.BlockSpec(memory_space=pltpu.SMEM),
            pl.BlockSpec((1, ts, dim), lambda b, s: (b, s, 0)),
            pl.BlockSpec((1, 1, dim), lambda b, s: (b, 0, 0)),
        ],
        out_specs=[
            pl.BlockSpec((1, 1, ts), lambda b, s: (b * ns + s, 0, 0)),
            pl.BlockSpec((1, 1, dim), lambda b, s: (b, 0, 0)),
        ],
        out_shape=[
            jax.ShapeDtypeStruct((batch * ns, 1, ts), jnp.float32),
            jax.ShapeDtypeStruct((batch, 1, dim), jnp.float32),
        ],
        scratch_shapes=[
            pltpu.SMEM((2,), jnp.int32),
            pltpu.SMEM((2,), jnp.float32),
            pltpu.VMEM((1, dim), jnp.float32),
        ],
    )(lens, sel, context, tgt3)
    return (weighted.reshape(batch, dim), attn.reshape(batch, seq))


# speedup vs baseline: 1.4975x; 1.0540x over previous
import jax
import jax.numpy as jnp
from jax.experimental import pallas as pl
from jax.experimental.pallas import tpu as pltpu

_NEG = -1e30


def _target_kernel(h_ref, w_ref, out_ref):
    out_ref[...] = jax.lax.dot_general(
        h_ref[...], w_ref[...], (((1,), (1,)), ((), ())),
        preferred_element_type=jnp.float32)


def _attn_kernel(lens_ref, sel_ref, ctx_ref, tgt_ref, attn_ref, wout_ref,
                 scl_ref, md_ref, acc_ref):
    b = pl.program_id(0)
    s = pl.program_id(1)
    ns = pl.num_programs(1)
    ts = ctx_ref.shape[1]
    nblk = lens_ref.shape[1]

    @pl.when(s == 0)
    def _init():
        sel = sel_ref[b]

        def body(j, tot):
            return tot + jnp.where(j < sel, lens_ref[b, j], 0)

        scl_ref[0] = jax.lax.fori_loop(0, nblk, body, 0) + 1
        scl_ref[1] = lens_ref[b, sel]
        md_ref[0] = _NEG
        md_ref[1] = 0.0
        acc_ref[...] = jnp.zeros_like(acc_ref)

    ctx = ctx_ref[0]
    tgt = tgt_ref[0]
    attn_row = jax.lax.dot_general(
        tgt, ctx, (((1,), (1,)), ((), ())),
        preferred_element_type=jnp.float32)
    attn_ref[0] = attn_row

    sc = scl_ref[0]
    ln = scl_ref[1]
    t0 = s * ts
    overlap = (t0 < sc + ln) & (t0 + ts > sc)

    @pl.when(overlap)
    def _update():
        pos = t0 + jax.lax.broadcasted_iota(jnp.int32, (1, ts), 1)
        mask = (pos >= sc) & (pos < sc + ln)
        masked = jnp.where(mask, attn_row, _NEG)
        m_old = md_ref[0]
        m_new = jnp.maximum(m_old, jnp.max(masked))
        scale = jnp.exp(m_old - m_new)
        unnorm = jnp.where(mask, jnp.exp(masked - m_new), 0.0)
        md_ref[0] = m_new
        md_ref[1] = md_ref[1] * scale + jnp.sum(unnorm)
        acc_ref[...] = acc_ref[...] * scale + jax.lax.dot_general(
            unnorm, ctx, (((1,), (0,)), ((), ())),
            preferred_element_type=jnp.float32)

    @pl.when(s == ns - 1)
    def _finalize():
        d = md_ref[1]
        denom = jnp.where(d == 0.0, 1.0, d)
        wout_ref[0] = acc_ref[...] / denom


def kernel(h, context, sub_seq_lengths, selected_block_idx, W):
    batch, dim = h.shape
    seq = context.shape[1]

    td = 1024
    target = pl.pallas_call(
        _target_kernel,
        grid=(dim // td,),
        in_specs=[
            pl.BlockSpec((batch, dim), lambda i: (0, 0)),
            pl.BlockSpec((td, dim), lambda i: (i, 0)),
        ],
        out_specs=pl.BlockSpec((batch, td), lambda i: (0, i)),
        out_shape=jax.ShapeDtypeStruct((batch, dim), jnp.float32),
    )(h, W)

    ts = 512
    ns = seq // ts
    lens = sub_seq_lengths.astype(jnp.int32)
    sel = selected_block_idx.astype(jnp.int32)
    tgt3 = target.reshape(batch, 1, dim)
    attn, weighted = pl.pallas_call(
        _attn_kernel,
        grid=(batch, ns),
        in_specs=[
            pl.BlockSpec(memory_space=pltpu.SMEM),
            pl.BlockSpec(memory_space=pltpu.SMEM),
            pl.BlockSpec((1, ts, dim), lambda b, s: (b, s, 0)),
            pl.BlockSpec((1, 1, dim), lambda b, s: (b, 0, 0)),
        ],
        out_specs=[
            pl.BlockSpec((1, 1, ts), lambda b, s: (b * ns + s, 0, 0)),
            pl.BlockSpec((1, 1, dim), lambda b, s: (b, 0, 0)),
        ],
        out_shape=[
            jax.ShapeDtypeStruct((batch * ns, 1, ts), jnp.float32),
            jax.ShapeDtypeStruct((batch, 1, dim), jnp.float32),
        ],
        scratch_shapes=[
            pltpu.SMEM((2,), jnp.int32),
            pltpu.SMEM((2,), jnp.float32),
            pltpu.VMEM((1, dim), jnp.float32),
        ],
    )(lens, sel, context, tgt3)
    return (weighted.reshape(batch, dim), attn.reshape(batch, seq))


# speedup vs baseline: 1.5406x; 1.0288x over previous
import jax
import jax.numpy as jnp
from jax.experimental import pallas as pl
from jax.experimental.pallas import tpu as pltpu

_NEG = -1e30


def _target_kernel(h_ref, w_ref, out_ref):
    out_ref[...] = jax.lax.dot_general(
        h_ref[...], w_ref[...], (((1,), (1,)), ((), ())),
        preferred_element_type=jnp.float32)


def _attn_kernel(lens_ref, sel_ref, ctx_ref, tgt_ref, attn_ref, wout_ref,
                 scl_ref, md_ref, acc_ref):
    b = pl.program_id(0)
    s = pl.program_id(1)
    ns = pl.num_programs(1)
    ts = ctx_ref.shape[1]
    nblk = lens_ref.shape[1]

    @pl.when(s == 0)
    def _init():
        sel = sel_ref[b]

        def body(j, tot):
            return tot + jnp.where(j < sel, lens_ref[b, j], 0)

        scl_ref[0] = jax.lax.fori_loop(0, nblk, body, 0) + 1
        scl_ref[1] = lens_ref[b, sel]
        md_ref[0] = _NEG
        md_ref[1] = 0.0
        acc_ref[...] = jnp.zeros_like(acc_ref)

    ctx = ctx_ref[0]
    tgt = tgt_ref[0]
    attn_row = jax.lax.dot_general(
        tgt, ctx, (((1,), (1,)), ((), ())),
        preferred_element_type=jnp.float32)
    attn_ref[0] = attn_row

    sc = scl_ref[0]
    ln = scl_ref[1]
    t0 = s * ts
    overlap = (t0 < sc + ln) & (t0 + ts > sc)

    @pl.when(overlap)
    def _update():
        pos = t0 + jax.lax.broadcasted_iota(jnp.int32, (1, ts), 1)
        mask = (pos >= sc) & (pos < sc + ln)
        masked = jnp.where(mask, attn_row, _NEG)
        m_old = md_ref[0]
        m_new = jnp.maximum(m_old, jnp.max(masked))
        scale = jnp.exp(m_old - m_new)
        unnorm = jnp.where(mask, jnp.exp(masked - m_new), 0.0)
        md_ref[0] = m_new
        md_ref[1] = md_ref[1] * scale + jnp.sum(unnorm)
        acc_ref[...] = acc_ref[...] * scale + jax.lax.dot_general(
            unnorm, ctx, (((1,), (0,)), ((), ())),
            preferred_element_type=jnp.float32)

    @pl.when(s == ns - 1)
    def _finalize():
        d = md_ref[1]
        denom = jnp.where(d == 0.0, 1.0, d)
        wout_ref[0] = acc_ref[...] / denom


def kernel(h, context, sub_seq_lengths, selected_block_idx, W):
    batch, dim = h.shape
    seq = context.shape[1]

    td = 1024
    target = pl.pallas_call(
        _target_kernel,
        grid=(dim // td,),
        in_specs=[
            pl.BlockSpec((batch, dim), lambda i: (0, 0)),
            pl.BlockSpec((td, dim), lambda i: (i, 0)),
        ],
        out_specs=pl.BlockSpec((batch, td), lambda i: (0, i)),
        out_shape=jax.ShapeDtypeStruct((batch, dim), jnp.float32),
    )(h, W)

    ts = 1024
    ns = seq // ts
    lens = sub_seq_lengths.astype(jnp.int32)
    sel = selected_block_idx.astype(jnp.int32)
    tgt3 = target.reshape(batch, 1, dim)
    attn, weighted = pl.pallas_call(
        _attn_kernel,
        grid=(batch, ns),
        in_specs=[
            pl.BlockSpec(memory_space=pltpu.SMEM),
            pl.BlockSpec(memory_space=pltpu.SMEM),
            pl.BlockSpec((1, ts, dim), lambda b, s: (b, s, 0)),
            pl.BlockSpec((1, 1, dim), lambda b, s: (b, 0, 0)),
        ],
        out_specs=[
            pl.BlockSpec((1, 1, ts), lambda b, s: (b * ns + s, 0, 0)),
            pl.BlockSpec((1, 1, dim), lambda b, s: (b, 0, 0)),
        ],
        out_shape=[
            jax.ShapeDtypeStruct((batch * ns, 1, ts), jnp.float32),
            jax.ShapeDtypeStruct((batch, 1, dim), jnp.float32),
        ],
        scratch_shapes=[
            pltpu.SMEM((2,), jnp.int32),
            pltpu.SMEM((2,), jnp.float32),
            pltpu.VMEM((1, dim), jnp.float32),
        ],
    )(lens, sel, context, tgt3)
    return (weighted.reshape(batch, dim), attn.reshape(batch, seq))


# speedup vs baseline: 1.5409x; 1.0002x over previous
import jax
import jax.numpy as jnp
from jax.experimental import pallas as pl
from jax.experimental.pallas import tpu as pltpu

_NEG = -1e30


def _target_kernel(h_ref, w_ref, out_ref):
    out_ref[...] = jax.lax.dot_general(
        h_ref[...], w_ref[...], (((1,), (1,)), ((), ())),
        preferred_element_type=jnp.float32)


def _attn_kernel(lens_ref, sel_ref, ctx_ref, ctx2_ref, tgt_ref, attn_ref,
                 attn2_ref, wout_ref, scl_ref, md_ref, acc_ref):
    b = pl.program_id(0)
    s = pl.program_id(1)
    ns = pl.num_programs(1)
    hts = ctx_ref.shape[1]
    nblk = lens_ref.shape[1]

    @pl.when(s == 0)
    def _init():
        sel = sel_ref[b]

        def body(j, tot):
            return tot + jnp.where(j < sel, lens_ref[b, j], 0)

        scl_ref[0] = jax.lax.fori_loop(0, nblk, body, 0) + 1
        scl_ref[1] = lens_ref[b, sel]
        md_ref[0] = _NEG
        md_ref[1] = 0.0
        acc_ref[...] = jnp.zeros_like(acc_ref)

    tgt = tgt_ref[0]
    sc = scl_ref[0]
    ln = scl_ref[1]

    def half(cref, aref, t0):
        ctx = cref[0]
        attn_row = jax.lax.dot_general(
            tgt, ctx, (((1,), (1,)), ((), ())),
            preferred_element_type=jnp.float32)
        aref[0] = attn_row
        overlap = (t0 < sc + ln) & (t0 + hts > sc)

        @pl.when(overlap)
        def _update():
            pos = t0 + jax.lax.broadcasted_iota(jnp.int32, (1, hts), 1)
            mask = (pos >= sc) & (pos < sc + ln)
            masked = jnp.where(mask, attn_row, _NEG)
            m_old = md_ref[0]
            m_new = jnp.maximum(m_old, jnp.max(masked))
            scale = jnp.exp(m_old - m_new)
            unnorm = jnp.where(mask, jnp.exp(masked - m_new), 0.0)
            md_ref[0] = m_new
            md_ref[1] = md_ref[1] * scale + jnp.sum(unnorm)
            acc_ref[...] = acc_ref[...] * scale + jax.lax.dot_general(
                unnorm, ctx, (((1,), (0,)), ((), ())),
                preferred_element_type=jnp.float32)

    half(ctx_ref, attn_ref, s * 2 * hts)
    half(ctx2_ref, attn2_ref, s * 2 * hts + hts)

    @pl.when(s == ns - 1)
    def _finalize():
        d = md_ref[1]
        denom = jnp.where(d == 0.0, 1.0, d)
        wout_ref[0] = acc_ref[...] / denom


def kernel(h, context, sub_seq_lengths, selected_block_idx, W):
    batch, dim = h.shape
    seq = context.shape[1]

    td = 1024
    target = pl.pallas_call(
        _target_kernel,
        grid=(dim // td,),
        in_specs=[
            pl.BlockSpec((batch, dim), lambda i: (0, 0)),
            pl.BlockSpec((td, dim), lambda i: (i, 0)),
        ],
        out_specs=pl.BlockSpec((batch, td), lambda i: (0, i)),
        out_shape=jax.ShapeDtypeStruct((batch, dim), jnp.float32),
    )(h, W)

    ts = 1024
    hts = ts // 2
    ns = seq // ts
    lens = sub_seq_lengths.astype(jnp.int32)
    sel = selected_block_idx.astype(jnp.int32)
    tgt3 = target.reshape(batch, 1, dim)
    attn_a, attn_b, weighted = pl.pallas_call(
        _attn_kernel,
        grid=(batch, ns),
        in_specs=[
            pl.BlockSpec(memory_space=pltpu.SMEM),
            pl.BlockSpec(memory_space=pltpu.SMEM),
            pl.BlockSpec((1, hts, dim), lambda b, s: (b, 2 * s, 0)),
            pl.BlockSpec((1, hts, dim), lambda b, s: (b, 2 * s + 1, 0)),
            pl.BlockSpec((1, 1, dim), lambda b, s: (b, 0, 0)),
        ],
        out_specs=[
            pl.BlockSpec((1, 1, hts), lambda b, s: (b * ns + s, 0, 0)),
            pl.BlockSpec((1, 1, hts), lambda b, s: (b * ns + s, 0, 0)),
            pl.BlockSpec((1, 1, dim), lambda b, s: (b, 0, 0)),
        ],
        out_shape=[
            jax.ShapeDtypeStruct((batch * ns, 1, hts), jnp.float32),
            jax.ShapeDtypeStruct((batch * ns, 1, hts), jnp.float32),
            jax.ShapeDtypeStruct((batch, 1, dim), jnp.float32),
        ],
        scratch_shapes=[
            pltpu.SMEM((2,), jnp.int32),
            pltpu.SMEM((2,), jnp.float32),
            pltpu.VMEM((1, dim), jnp.float32),
        ],
    )(lens, sel, context, context, tgt3)
    attn = jnp.concatenate(
        [attn_a.reshape(batch, ns, 1, hts), attn_b.reshape(batch, ns, 1, hts)],
        axis=2).reshape(batch, seq)
    return (weighted.reshape(batch, dim), attn)
